# Initial kernel scaffold; baseline (speedup 1.0000x reference)
import jax
import jax.numpy as jnp
from jax.experimental import pallas as pl


def kernel(features, edge_indices, edge_weights, params):
    raise NotImplementedError("write your pallas kernel here")


# speedup vs baseline: 2.9853x; 2.9853x over previous
import jax
import jax.numpy as jnp
from jax import lax
from jax.experimental import pallas as pl
from jax.experimental.pallas import tpu as pltpu
from jax.experimental.pallas import tpu_sc as plsc

N = 10000
E = 320000
R = 3
D = 128
H = 128

NC = 2
NS = 16
CH = 128
EC = E // CH
RPS = N // NS
ZB = 125


def _tca_body(x_ref, pW, pb, a0W1, a0b1, a0W2, a0b2, a1W1, a1b1, a1W2, a1b2,
              rwW, rwb, rt0W, rt0b, rt1W, rt1b, rt2W, rt2b,
              slW, slb, ftW, ftb,
              y0, y1, y2, so, tr, narw):
    x = x_ref[...]
    cl = x @ pW[...] + pb[...]
    cl = cl - jnp.max(cl, axis=-1, keepdims=True)
    ecl = jnp.exp(cl)
    cp = ecl / jnp.sum(ecl, axis=-1, keepdims=True)
    s0 = jnp.maximum(x @ a0W1[...] + a0b1[...], 0.0) @ a0W2[...] + a0b2[...]
    s1 = jnp.maximum(x @ a1W1[...] + a1b1[...], 0.0) @ a1W2[...] + a1b2[...]
    natt = s0 * cp[:, 0:1] + s1 * cp[:, 1:2]
    rl = x @ rwW[...] + rwb[...]
    rl = rl - jnp.max(rl, axis=-1, keepdims=True)
    erl = jnp.exp(rl)
    rw = erl / jnp.sum(erl, axis=-1, keepdims=True)
    y0[...] = x @ rt0W[...] + rt0b[...]
    y1[...] = x @ rt1W[...] + rt1b[...]
    y2[...] = x @ rt2W[...] + rt2b[...]
    so[...] = x @ slW[...] + slb[...]
    tr[...] = x @ ftW[...] + ftb[...]
    narw[:, 0:1] = natt
    narw[:, 1:4] = rw


def _run_tca(x, p, l):
    BN = 1000
    grid = (N // BN,)
    row = pl.BlockSpec((BN, D), lambda i: (i, 0))
    full = lambda a: pl.BlockSpec(a.shape, lambda i: (0,) * a.ndim)
    w = [p[f'l{l}_pred_W'], p[f'l{l}_pred_b'],
         p[f'l{l}_att0_W1'], p[f'l{l}_att0_b1'], p[f'l{l}_att0_W2'], p[f'l{l}_att0_b2'],
         p[f'l{l}_att1_W1'], p[f'l{l}_att1_b1'], p[f'l{l}_att1_W2'], p[f'l{l}_att1_b2'],
         p[f'l{l}_rw_W'], p[f'l{l}_rw_b'],
         p[f'l{l}_rt0_W'], p[f'l{l}_rt0_b'], p[f'l{l}_rt1_W'], p[f'l{l}_rt1_b'],
         p[f'l{l}_rt2_W'], p[f'l{l}_rt2_b'],
         p[f'l{l}_sl_W'], p[f'l{l}_sl_b'], p[f'l{l}_ft_W'], p[f'l{l}_ft_b']]
    w = [a if a.ndim == 2 else a.reshape(1, -1) for a in w]
    fs = jax.ShapeDtypeStruct
    return pl.pallas_call(
        _tca_body,
        grid=grid,
        in_specs=[row] + [full(a) for a in w],
        out_specs=[row] * 5 + [pl.BlockSpec((BN, 4), lambda i: (i, 0))],
        out_shape=[fs((N, H), jnp.float32)] * 5 + [fs((N, 4), jnp.float32)],
    )(x, *w)


def _make_tcb_body(final):
    def body(part_ref, narw_ref, so_ref, tr_ref, gW, gb, fusWa, fusWb, fusb,
             lng, lnb, clsW, clsb, out_ref):
        part = part_ref[...]
        narw = narw_ref[...]
        combined = (narw[:, 1:2] * (part[0] + part[3]) +
                    narw[:, 2:3] * (part[1] + part[4]) +
                    narw[:, 3:4] * (part[2] + part[5]))
        gate = 1.0 / (1.0 + jnp.exp(-(combined @ gW[...] + gb[...])))
        weighted = gate * combined * narw[:, 0:1]
        fused = jnp.maximum(
            so_ref[...] @ fusWa[...] + weighted @ fusWb[...] + fusb[...], 0.0)
        o = fused + tr_ref[...]
        mu = jnp.mean(o, axis=-1, keepdims=True)
        var = jnp.mean((o - mu) ** 2, axis=-1, keepdims=True)
        o = (o - mu) * lax.rsqrt(var + 1e-5) * lng[...] + lnb[...]
        if final:
            o = o @ clsW[...] + clsb[...]
        out_ref[...] = o
    return body


def _run_tcb(part, narw, so, tr, p, l, final):
    BN = 1000
    grid = (N // BN,)
    row = lambda k: pl.BlockSpec((BN, k), lambda i: (i, 0))
    prow = pl.BlockSpec((2 * R, BN, H), lambda i: (0, i, 0))
    full = lambda a: pl.BlockSpec(a.shape, lambda i: (0,) * a.ndim)
    fusW = p[f'l{l}_fus_W']
    w = [p[f'l{l}_gate_W'], p[f'l{l}_gate_b'], fusW[:H], fusW[H:],
         p[f'l{l}_fus_b'], p[f'l{l}_ln_g'], p[f'l{l}_ln_b'],
         p['cls_W'], p['cls_b']]
    w = [a if a.ndim == 2 else a.reshape(1, -1) for a in w]
    dout = 2 if final else H
    return pl.pallas_call(
        _make_tcb_body(final),
        grid=grid,
        in_specs=[prow, row(4), row(H), row(H)] + [full(a) for a in w],
        out_specs=row(dout),
        out_shape=jax.ShapeDtypeStruct((N, dout), jnp.float32),
    )(part, narw, so, tr, *w)


def _sc_body(y0, y1, y2, ei, ew, out, acc, zbuf, sidx, didx, ewb, rows, gsem):
    c = lax.axis_index("c")
    s = lax.axis_index("s")
    zv = jnp.zeros((16,), jnp.float32)

    def zrow(i, _):
        for k in range(8):
            zbuf[i, pl.ds(k * 16, 16)] = zv
        return 0
    lax.fori_loop(0, ZB, zrow, 0)

    ys = (y0, y1, y2)
    base = c * (EC // NC)
    lo = base + (EC // NC) * s // NS
    hi = base + (EC // NC) * (s + 1) // NS

    for r in range(R):
        for k in range(RPS // ZB):
            pltpu.sync_copy(zbuf, acc.at[pl.ds(s * RPS + k * ZB, ZB)])
        plsc.subcore_barrier()

        def chunk(j, _):
            pltpu.sync_copy(ei.at[r, 0, j], sidx)
            pltpu.sync_copy(ei.at[r, 1, j], didx)
            pltpu.sync_copy(ew.at[r, j], ewb)
            pltpu.async_copy(ys[r].at[sidx], rows, gsem).wait()

            def row(e, _):
                wv = plsc.load_gather(ewb, [jnp.full((16,), e, jnp.int32)])
                for k in range(8):
                    sl = pl.ds(k * 16, 16)
                    rows[e, sl] = rows[e, sl] * wv
                return 0
            lax.fori_loop(0, CH, row, 0)
            pltpu.sync_copy(rows, acc.at[didx], add=True)
            return 0
        lax.fori_loop(lo, hi, chunk, 0)
        plsc.subcore_barrier()
        pltpu.sync_copy(acc.at[pl.ds(s * RPS, RPS)],
                        out.at[c * R + r, pl.ds(s * RPS, RPS)])
        plsc.subcore_barrier()


def _run_sc(y0, y1, y2, ei, ew):
    mesh = plsc.VectorSubcoreMesh(core_axis_name="c", subcore_axis_name="s",
                                  num_cores=NC, num_subcores=NS)
    f = pl.kernel(
        _sc_body,
        out_type=jax.ShapeDtypeStruct((2 * R, N, H), jnp.float32),
        mesh=mesh,
        compiler_params=pltpu.CompilerParams(use_tc_tiling_on_sc=False,
                                             needs_layout_passes=False),
        scratch_types=[
            pltpu.VMEM_SHARED((N, H), jnp.float32),
            pltpu.VMEM((ZB, H), jnp.float32),
            pltpu.VMEM((CH,), jnp.int32),
            pltpu.VMEM((CH,), jnp.int32),
            pltpu.VMEM((CH,), jnp.float32),
            pltpu.VMEM((CH, H), jnp.float32),
            pltpu.SemaphoreType.DMA,
        ],
    )
    return f(y0, y1, y2, ei, ew)


def kernel(features, edge_indices, edge_weights, params):
    ei = edge_indices.reshape(R, 2, EC, CH)
    ew = edge_weights.reshape(R, EC, CH)
    x = features
    for l in range(2):
        y0, y1, y2, so, tr, narw = _run_tca(x, params, l)
        part = _run_sc(y0, y1, y2, ei, ew)
        x = _run_tcb(part, narw, so, tr, params, l, final=(l == 1))
    return x
